# baseline (speedup 1.0000x reference)
import functools

import jax
import jax.numpy as jnp
from jax.experimental import pallas as pl
from jax.experimental.pallas import tpu as pltpu

BM = 128
BK = 16384


def _mm_kernel(adj_ref, w_ref, b_ref, o_ref, acc_ref):
    j = pl.program_id(1)

    @pl.when(j == 0)
    def _init():
        acc_ref[...] = jnp.zeros_like(acc_ref)

    acc_ref[...] += jnp.dot(
        adj_ref[...],
        w_ref[pl.ds(j * BK, BK), :],
        preferred_element_type=jnp.float32,
    )

    @pl.when(j == pl.num_programs(1) - 1)
    def _store():
        o_ref[...] = acc_ref[...] + b_ref[...]


@jax.jit
def kernel(adj, W, b):
    n, k = adj.shape
    out_f = W.shape[1]
    b2 = b.reshape(1, out_f)
    grid = (n // BM, k // BK)
    return pl.pallas_call(
        _mm_kernel,
        grid=grid,
        in_specs=[
            pl.BlockSpec((BM, BK), lambda i, j: (i, j)),
            pl.BlockSpec((k, out_f), lambda i, j: (0, 0)),
            pl.BlockSpec((1, out_f), lambda i, j: (0, 0)),
        ],
        out_specs=pl.BlockSpec((BM, out_f), lambda i, j: (i, 0)),
        out_shape=jax.ShapeDtypeStruct((n, out_f), jnp.float32),
        scratch_shapes=[pltpu.VMEM((BM, out_f), jnp.float32)],
        compiler_params=pltpu.CompilerParams(
            dimension_semantics=("parallel", "arbitrary"),
        ),
    )(adj, W, b2)


# speedup vs baseline: 1.0060x; 1.0060x over previous
import functools

import jax
import jax.numpy as jnp
from jax.experimental import pallas as pl
from jax.experimental.pallas import tpu as pltpu

BM = 256
BK = 16384


def _mm_kernel(adj_ref, w_ref, b_ref, o_ref, acc_ref):
    j = pl.program_id(1)

    @pl.when(j == 0)
    def _init():
        acc_ref[...] = jnp.zeros_like(acc_ref)

    acc_ref[...] += jnp.dot(
        adj_ref[...],
        w_ref[pl.ds(j * BK, BK), :],
        preferred_element_type=jnp.float32,
    )

    @pl.when(j == pl.num_programs(1) - 1)
    def _store():
        o_ref[...] = acc_ref[...] + b_ref[...]


@jax.jit
def kernel(adj, W, b):
    n, k = adj.shape
    out_f = W.shape[1]
    b2 = b.reshape(1, out_f)
    grid = (n // BM, k // BK)
    return pl.pallas_call(
        _mm_kernel,
        grid=grid,
        in_specs=[
            pl.BlockSpec((BM, BK), lambda i, j: (i, j)),
            pl.BlockSpec((k, out_f), lambda i, j: (0, 0)),
            pl.BlockSpec((1, out_f), lambda i, j: (0, 0)),
        ],
        out_specs=pl.BlockSpec((BM, out_f), lambda i, j: (i, 0)),
        out_shape=jax.ShapeDtypeStruct((n, out_f), jnp.float32),
        scratch_shapes=[pltpu.VMEM((BM, out_f), jnp.float32)],
        compiler_params=pltpu.CompilerParams(
            dimension_semantics=("parallel", "arbitrary"),
        ),
    )(adj, W, b2)
